# Initial kernel scaffold; baseline (speedup 1.0000x reference)
import jax
import jax.numpy as jnp
from jax.experimental import pallas as pl


def kernel(uv, feat_3d, w1, b1, w2, b2, w_out, b_out, image_h, image_w):
    raise NotImplementedError("write your pallas kernel here")


# speedup vs baseline: 25.9233x; 25.9233x over previous
import jax
import jax.numpy as jnp
from jax.experimental import pallas as pl

_H, _W = 60, 80
_HW = _H * _W
_QB = 192
_K = 3


def _fwd_body(gxy_ref, uv_ref, ftab_ref, w1_ref, b1_ref, w2t_ref, b2_ref,
              wot_ref, bo_ref, out_ref):
    N = uv_ref.shape[2]
    C = w2t_ref.shape[1]
    qx = gxy_ref[:, 0:1]
    qy = gxy_ref[:, 1:2]
    px = uv_ref[0, 0:1, :]
    py = uv_ref[0, 1:2, :]
    q2 = qx * qx + qy * qy
    p2 = px * px + py * py
    qp = jnp.dot(gxy_ref[:, :], uv_ref[0],
                 preferred_element_type=jnp.float32)
    d = q2 - 2.0 * qp + p2

    li = jax.lax.broadcasted_iota(jnp.int32, (_QB, N), 1)
    ftab = ftab_ref[0]
    acc = jnp.zeros((_QB, C), jnp.float32)
    for k in range(_K):
        m = jnp.min(d, axis=1, keepdims=True)
        eq = d == m
        idxk = jnp.min(jnp.where(eq, li, N), axis=1,
                       keepdims=True)
        sel = li == idxk
        oh = jnp.where(sel, 1.0, 0.0)
        g = jnp.dot(oh, ftab, preferred_element_type=jnp.float32,
                    precision=jax.lax.Precision.HIGHEST)
        if k < _K - 1:
            d = jnp.where(sel, jnp.inf, d)
        offx = g[:, 0:1] - qx
        offy = g[:, 1:2] - qy
        nrm = jnp.sqrt(offx * offx + offy * offy)
        logits = b2_ref[0:1, :]
        for j in range(3):
            h = (offx * w1_ref[j:j + 1, 0:1] + offy * w1_ref[j:j + 1, 1:2]
                 + nrm * w1_ref[j:j + 1, 2:3] + b1_ref[0:1, j:j + 1])
            h = jnp.where(h >= 0, h, 0.1 * h)
            logits = logits + h * w2t_ref[j:j + 1, :]
        score = jax.nn.sigmoid(logits)
        acc = acc + score * g[:, 2:]

    o = jnp.dot(acc, wot_ref[:, :], preferred_element_type=jnp.float32,
                precision=jax.lax.Precision.HIGHEST) + bo_ref[0:1, :]
    out_ref[0] = jnp.where(o >= 0, o, 0.1 * o)


def kernel(uv, feat_3d, w1, b1, w2, b2, w_out, b_out, image_h, image_w):
    bs, _, N = uv.shape
    C = feat_3d.shape[1]
    T = _HW // _QB

    r = ((jnp.asarray(image_h, jnp.float32) - _H)
         + (jnp.asarray(image_w, jnp.float32) - _W))
    idx = jnp.arange(_HW, dtype=jnp.int32)
    xs = (idx % _W).astype(jnp.float32)
    ys = (idx // _W).astype(jnp.float32)
    gxy = jnp.stack([xs, ys], axis=1) + r
    ftab = jnp.concatenate([jnp.swapaxes(uv, 1, 2),
                            jnp.swapaxes(feat_3d, 1, 2)], axis=2)

    out_qm = pl.pallas_call(
        _fwd_body,
        grid=(bs, T),
        in_specs=[
            pl.BlockSpec((_QB, 2), lambda b, t: (t, 0)),
            pl.BlockSpec((1, 2, N), lambda b, t: (b, 0, 0)),
            pl.BlockSpec((1, N, 2 + C), lambda b, t: (b, 0, 0)),
            pl.BlockSpec((3, 3), lambda b, t: (0, 0)),
            pl.BlockSpec((1, 3), lambda b, t: (0, 0)),
            pl.BlockSpec((3, C), lambda b, t: (0, 0)),
            pl.BlockSpec((1, C), lambda b, t: (0, 0)),
            pl.BlockSpec((C, C), lambda b, t: (0, 0)),
            pl.BlockSpec((1, C), lambda b, t: (0, 0)),
        ],
        out_specs=pl.BlockSpec((1, _QB, C), lambda b, t: (b, t, 0)),
        out_shape=jax.ShapeDtypeStruct((bs, _HW, C), jnp.float32),
    )(gxy, uv, ftab, w1, b1[None, :], w2.T, b2[None, :], w_out.T,
      b_out[None, :])

    return jnp.swapaxes(out_qm, 1, 2).reshape(bs, C, _H, _W)


# speedup vs baseline: 54.3900x; 2.0981x over previous
import functools

import jax
import jax.numpy as jnp
from jax import lax
from jax.experimental import pallas as pl
from jax.experimental.pallas import tpu as pltpu
from jax.experimental.pallas import tpu_sc as plsc

_H, _W = 60, 80
_HW = _H * _W
_QB = 192
_K = 3
_D = 128
_NC, _NS = 2, 16


def _knn_body(gxy_ref, uv_ref, idx_ref):
    N = uv_ref.shape[2]
    b = pl.program_id(0)
    qx = gxy_ref[:, 0:1]
    qy = gxy_ref[:, 1:2]
    px = uv_ref[0, 0:1, :]
    py = uv_ref[0, 1:2, :]
    q2 = qx * qx + qy * qy
    p2 = px * px + py * py
    qp = jnp.dot(gxy_ref[:, :], uv_ref[0],
                 preferred_element_type=jnp.float32)
    d = q2 - 2.0 * qp + p2

    li = jax.lax.broadcasted_iota(jnp.int32, (_QB, N), 1)
    cols = []
    for k in range(_K):
        m = jnp.min(d, axis=1, keepdims=True)
        eq = d == m
        idxk = jnp.min(jnp.where(eq, li, N), axis=1,
                       keepdims=True)
        cols.append(idxk)
        if k < _K - 1:
            d = jnp.where(li == idxk, jnp.inf, d)
    idx_ref[0] = jnp.concatenate(cols, axis=1) + b * N


def _interp_body(gxy_ref, rows_ref, w1_ref, b1_ref, w2t_ref, b2_ref,
                 wot_ref, bo_ref, out_ref):
    C = w2t_ref.shape[1]
    qx = gxy_ref[:, 0:1]
    qy = gxy_ref[:, 1:2]
    g = rows_ref[0]
    acc = jnp.zeros((_QB, C), jnp.float32)
    for k in range(_K):
        o = k * _D
        offx = g[:, o:o + 1] - qx
        offy = g[:, o + 1:o + 2] - qy
        nrm = jnp.sqrt(offx * offx + offy * offy)
        logits = b2_ref[0:1, :]
        for j in range(3):
            h = (offx * w1_ref[j:j + 1, 0:1] + offy * w1_ref[j:j + 1, 1:2]
                 + nrm * w1_ref[j:j + 1, 2:3] + b1_ref[0:1, j:j + 1])
            h = jnp.where(h >= 0, h, 0.1 * h)
            logits = logits + h * w2t_ref[j:j + 1, :]
        score = jax.nn.sigmoid(logits)
        acc = acc + score * g[:, o + 2:o + 2 + C]
    o = jnp.dot(acc, wot_ref[:, :], preferred_element_type=jnp.float32,
                precision=jax.lax.Precision.HIGHEST) + bo_ref[0:1, :]
    out_ref[0] = jnp.where(o >= 0, o, 0.1 * o)


def _sc_gather(table, idx_flat, b_per_w):
    nw = _NC * _NS
    mesh = plsc.VectorSubcoreMesh(core_axis_name="c", subcore_axis_name="s")

    @functools.partial(
        pl.kernel, mesh=mesh,
        out_type=jax.ShapeDtypeStruct((b_per_w * nw, _D), jnp.float32),
        scratch_types=[
            pltpu.VMEM((b_per_w,), jnp.int32),
            pltpu.VMEM((b_per_w, _D), jnp.float32),
            pltpu.SemaphoreType.DMA,
        ],
    )
    def k(idx_hbm, table_hbm, out_hbm, idx_v, rows_v, sem):
        wid = lax.axis_index("s") * _NC + lax.axis_index("c")
        base = wid * b_per_w
        pltpu.sync_copy(idx_hbm.at[pl.ds(base, b_per_w)], idx_v)
        pltpu.async_copy(table_hbm.at[idx_v], rows_v, sem).wait()
        pltpu.sync_copy(rows_v, out_hbm.at[pl.ds(base, b_per_w)])

    return k(idx_flat, table)


def kernel(uv, feat_3d, w1, b1, w2, b2, w_out, b_out, image_h, image_w):
    bs, _, N = uv.shape
    C = feat_3d.shape[1]
    T = _HW // _QB

    r = ((jnp.asarray(image_h, jnp.float32) - _H)
         + (jnp.asarray(image_w, jnp.float32) - _W))
    idx = jnp.arange(_HW, dtype=jnp.int32)
    xs = (idx % _W).astype(jnp.float32)
    ys = (idx // _W).astype(jnp.float32)
    gxy = jnp.stack([xs, ys], axis=1) + r

    knn_idx = pl.pallas_call(
        _knn_body,
        grid=(bs, T),
        in_specs=[
            pl.BlockSpec((_QB, 2), lambda b, t: (t, 0)),
            pl.BlockSpec((1, 2, N), lambda b, t: (b, 0, 0)),
        ],
        out_specs=pl.BlockSpec((1, _QB, _K), lambda b, t: (b, t, 0)),
        out_shape=jax.ShapeDtypeStruct((bs, _HW, _K), jnp.int32),
    )(gxy, uv)

    table = jnp.concatenate(
        [jnp.swapaxes(uv, 1, 2), jnp.swapaxes(feat_3d, 1, 2),
         jnp.zeros((bs, N, _D - 2 - C), jnp.float32)], axis=2,
    ).reshape(bs * N, _D)
    B = bs * _HW * _K
    nw = _NC * _NS
    b_pad = (B + 8 * nw - 1) // (8 * nw) * (8 * nw)
    idx_flat = jnp.concatenate(
        [knn_idx.reshape(B), jnp.zeros((b_pad - B,), jnp.int32)])
    rows = _sc_gather(table, idx_flat, b_pad // nw)
    rows = rows[:B].reshape(bs, _HW, _K * _D)

    out_qm = pl.pallas_call(
        _interp_body,
        grid=(bs, T),
        in_specs=[
            pl.BlockSpec((_QB, 2), lambda b, t: (t, 0)),
            pl.BlockSpec((1, _QB, _K * _D), lambda b, t: (b, t, 0)),
            pl.BlockSpec((3, 3), lambda b, t: (0, 0)),
            pl.BlockSpec((1, 3), lambda b, t: (0, 0)),
            pl.BlockSpec((3, C), lambda b, t: (0, 0)),
            pl.BlockSpec((1, C), lambda b, t: (0, 0)),
            pl.BlockSpec((C, C), lambda b, t: (0, 0)),
            pl.BlockSpec((1, C), lambda b, t: (0, 0)),
        ],
        out_specs=pl.BlockSpec((1, _QB, C), lambda b, t: (b, t, 0)),
        out_shape=jax.ShapeDtypeStruct((bs, _HW, C), jnp.float32),
    )(gxy, rows, w1, b1[None, :], w2.T, b2[None, :], w_out.T, b_out[None, :])

    return jnp.swapaxes(out_qm, 1, 2).reshape(bs, C, _H, _W)


# speedup vs baseline: 59.0677x; 1.0860x over previous
import functools

import jax
import jax.numpy as jnp
from jax import lax
from jax.experimental import pallas as pl
from jax.experimental.pallas import tpu as pltpu
from jax.experimental.pallas import tpu_sc as plsc

_H, _W = 60, 80
_HW = _H * _W
_QB = 192
_K = 3
_D = 128
_NC, _NS = 2, 16


def _knn_body(gxy_ref, uv_ref, idx_ref):
    N = uv_ref.shape[2]
    b = pl.program_id(0)
    qx = gxy_ref[:, 0:1]
    qy = gxy_ref[:, 1:2]
    px = uv_ref[0, 0:1, :]
    py = uv_ref[0, 1:2, :]
    q2 = qx * qx + qy * qy
    p2 = px * px + py * py
    qp = jnp.dot(gxy_ref[:, :], uv_ref[0],
                 preferred_element_type=jnp.float32)
    d = q2 - 2.0 * qp + p2

    li = jax.lax.broadcasted_iota(jnp.int32, (_QB, N), 1)
    cols = []
    for k in range(_K):
        m = jnp.min(d, axis=1, keepdims=True)
        eq = d == m
        idxk = jnp.min(jnp.where(eq, li, N), axis=1,
                       keepdims=True)
        cols.append(idxk)
        if k < _K - 1:
            d = jnp.where(li == idxk, jnp.inf, d)
    idx_ref[0] = jnp.concatenate(cols, axis=1) + b * N


def _interp_body(gxy_ref, g0_ref, g1_ref, g2_ref, w1_ref, b1_ref, w2t_ref,
                 b2_ref, wot_ref, bo_ref, out_ref):
    C = w2t_ref.shape[1]
    qx = gxy_ref[:, 0:1]
    qy = gxy_ref[:, 1:2]
    acc = jnp.zeros((_QB, C), jnp.float32)
    for g_ref in (g0_ref, g1_ref, g2_ref):
        g = g_ref[:, :]
        offx = g[:, 0:1] - qx
        offy = g[:, 1:2] - qy
        nrm = jnp.sqrt(offx * offx + offy * offy)
        logits = b2_ref[0:1, :]
        for j in range(3):
            h = (offx * w1_ref[j:j + 1, 0:1] + offy * w1_ref[j:j + 1, 1:2]
                 + nrm * w1_ref[j:j + 1, 2:3] + b1_ref[0:1, j:j + 1])
            h = jnp.where(h >= 0, h, 0.1 * h)
            logits = logits + h * w2t_ref[j:j + 1, :]
        score = jax.nn.sigmoid(logits)
        acc = acc + score * g[:, 2:2 + C]
    o = jnp.dot(acc, wot_ref[:, :], preferred_element_type=jnp.float32,
                precision=jax.lax.Precision.HIGHEST) + bo_ref[0:1, :]
    out_ref[0] = jnp.where(o >= 0, o, 0.1 * o)


def _sc_gather(table, idx_flat, b_per_w):
    nw = _NC * _NS
    mesh = plsc.VectorSubcoreMesh(core_axis_name="c", subcore_axis_name="s")

    @functools.partial(
        pl.kernel, mesh=mesh,
        out_type=jax.ShapeDtypeStruct((b_per_w * nw, _D), jnp.float32),
        scratch_types=[
            pltpu.VMEM((b_per_w,), jnp.int32),
            pltpu.VMEM((b_per_w, _D), jnp.float32),
            pltpu.SemaphoreType.DMA,
        ],
    )
    def k(idx_hbm, table_hbm, out_hbm, idx_v, rows_v, sem):
        wid = lax.axis_index("s") * _NC + lax.axis_index("c")
        base = wid * b_per_w
        pltpu.sync_copy(idx_hbm.at[pl.ds(base, b_per_w)], idx_v)
        pltpu.async_copy(table_hbm.at[idx_v], rows_v, sem).wait()
        pltpu.sync_copy(rows_v, out_hbm.at[pl.ds(base, b_per_w)])

    return k(idx_flat, table)


def kernel(uv, feat_3d, w1, b1, w2, b2, w_out, b_out, image_h, image_w):
    bs, _, N = uv.shape
    C = feat_3d.shape[1]
    T = _HW // _QB

    r = ((jnp.asarray(image_h, jnp.float32) - _H)
         + (jnp.asarray(image_w, jnp.float32) - _W))
    idx = jnp.arange(_HW, dtype=jnp.int32)
    xs = (idx % _W).astype(jnp.float32)
    ys = (idx // _W).astype(jnp.float32)
    gxy = jnp.stack([xs, ys], axis=1) + r

    knn_idx = pl.pallas_call(
        _knn_body,
        grid=(bs, T),
        in_specs=[
            pl.BlockSpec((_QB, 2), lambda b, t: (t, 0)),
            pl.BlockSpec((1, 2, N), lambda b, t: (b, 0, 0)),
        ],
        out_specs=pl.BlockSpec((1, _QB, _K), lambda b, t: (b, t, 0)),
        out_shape=jax.ShapeDtypeStruct((bs, _HW, _K), jnp.int32),
    )(gxy, uv)

    table = jnp.concatenate(
        [jnp.swapaxes(uv, 1, 2), jnp.swapaxes(feat_3d, 1, 2),
         jnp.zeros((bs, N, _D - 2 - C), jnp.float32)], axis=2,
    ).reshape(bs * N, _D)
    B = bs * _HW * _K
    nw = _NC * _NS
    b_pad = (B + 8 * nw - 1) // (8 * nw) * (8 * nw)
    idx_flat = jnp.concatenate(
        [jnp.transpose(knn_idx, (2, 0, 1)).reshape(B),
         jnp.zeros((b_pad - B,), jnp.int32)])
    rows = _sc_gather(table, idx_flat, b_pad // nw)

    Tb = bs * T
    out_qm = pl.pallas_call(
        _interp_body,
        grid=(bs, T),
        in_specs=[
            pl.BlockSpec((_QB, 2), lambda b, t: (t, 0)),
            pl.BlockSpec((_QB, _D), lambda b, t: (0 * Tb + b * T + t, 0)),
            pl.BlockSpec((_QB, _D), lambda b, t: (1 * Tb + b * T + t, 0)),
            pl.BlockSpec((_QB, _D), lambda b, t: (2 * Tb + b * T + t, 0)),
            pl.BlockSpec((3, 3), lambda b, t: (0, 0)),
            pl.BlockSpec((1, 3), lambda b, t: (0, 0)),
            pl.BlockSpec((3, C), lambda b, t: (0, 0)),
            pl.BlockSpec((1, C), lambda b, t: (0, 0)),
            pl.BlockSpec((C, C), lambda b, t: (0, 0)),
            pl.BlockSpec((1, C), lambda b, t: (0, 0)),
        ],
        out_specs=pl.BlockSpec((1, _QB, C), lambda b, t: (b, t, 0)),
        out_shape=jax.ShapeDtypeStruct((bs, _HW, C), jnp.float32),
    )(gxy, rows, rows, rows, w1, b1[None, :], w2.T, b2[None, :], w_out.T,
      b_out[None, :])

    return jnp.swapaxes(out_qm, 1, 2).reshape(bs, C, _H, _W)
